# baseline (speedup 1.0000x reference)
import functools

import jax
import jax.numpy as jnp
from jax import lax
from jax.experimental import pallas as pl
from jax.experimental.pallas import tpu as pltpu
from jax.experimental.pallas import tpu_sc as plsc

N = 10000
E = 320000
D = 128

NC = 2
NS = 16
NW = NC * NS

G = 128
K = 80
CK = 16
E_PAD = NW * K * G
N_PAD = 10240
ROWS_PER_TILE = N_PAD // NS


def _segment_sum_sc(m_pad, src_r, dst_r):
    mesh = plsc.VectorSubcoreMesh(core_axis_name="c", subcore_axis_name="s")

    @functools.partial(
        pl.kernel,
        out_type=jax.ShapeDtypeStruct((NC, N_PAD, D), jnp.float32),
        mesh=mesh,
        scratch_types=[
            pltpu.VMEM((CK, G), jnp.int32),
            pltpu.VMEM((CK, G), jnp.int32),
            pltpu.VMEM((G, D), jnp.float32),
            pltpu.VMEM((G, D), jnp.float32),
            pltpu.VMEM_SHARED((N_PAD, D), jnp.float32),
            pltpu.SemaphoreType.DMA,
            pltpu.SemaphoreType.DMA,
            pltpu.SemaphoreType.DMA,
            pltpu.SemaphoreType.DMA,
        ],
    )
    def k(m_hbm, src_hbm, dst_hbm, z_hbm, out_hbm, src_v, dst_v, bufa, bufb,
          acc, sema, semb, ssema, ssemb):
        cid = lax.axis_index("c")
        sid = lax.axis_index("s")
        wid = cid * NS + sid

        row0 = sid * ROWS_PER_TILE

        for r in range(0, ROWS_PER_TILE, G):
            pltpu.async_copy(z_hbm.at[pl.ds(row0 + r, G)],
                             acc.at[pl.ds(row0 + r, G)], sema)
        for r in range(0, ROWS_PER_TILE, G):
            pltpu.make_async_copy(z_hbm.at[pl.ds(row0 + r, G)],
                                  acc.at[pl.ds(row0 + r, G)], sema).wait()

        plsc.subcore_barrier()

        @pl.loop(0, K, step=CK)
        def _(c):
            pltpu.sync_copy(src_hbm.at[pl.ds(wid * K + c, CK)], src_v)
            pltpu.sync_copy(dst_hbm.at[pl.ds(wid * K + c, CK)], dst_v)

            pltpu.async_copy(m_hbm.at[src_v.at[0]], bufa, sema)

            @pl.loop(0, CK - 2, step=2)
            def _(j):
                pltpu.async_copy(m_hbm.at[src_v.at[j + 1]], bufb, semb)
                pltpu.make_async_copy(m_hbm.at[src_v.at[j]], bufa,
                                      sema).wait()
                pltpu.sync_copy(bufa, acc.at[dst_v.at[j]], add=True)
                pltpu.async_copy(m_hbm.at[src_v.at[j + 2]], bufa, sema)
                pltpu.make_async_copy(m_hbm.at[src_v.at[j + 1]], bufb,
                                      semb).wait()
                pltpu.sync_copy(bufb, acc.at[dst_v.at[j + 1]], add=True)

            pltpu.async_copy(m_hbm.at[src_v.at[CK - 1]], bufb, semb)
            pltpu.make_async_copy(m_hbm.at[src_v.at[CK - 2]], bufa,
                                  sema).wait()
            pltpu.sync_copy(bufa, acc.at[dst_v.at[CK - 2]], add=True)
            pltpu.make_async_copy(m_hbm.at[src_v.at[CK - 1]], bufb,
                                  semb).wait()
            pltpu.sync_copy(bufb, acc.at[dst_v.at[CK - 1]], add=True)

        plsc.subcore_barrier()

        for r in range(0, ROWS_PER_TILE, G):
            pltpu.async_copy(acc.at[pl.ds(row0 + r, G)],
                             out_hbm.at[cid, pl.ds(row0 + r, G)], semb)
        for r in range(0, ROWS_PER_TILE, G):
            pltpu.make_async_copy(acc.at[pl.ds(row0 + r, G)],
                                  out_hbm.at[cid, pl.ds(row0 + r, G)],
                                  semb).wait()

    return k(m_pad, src_r, dst_r, jnp.zeros((N_PAD, D), jnp.float32))


_BM = 256


def _dot_t(a, b):
    return lax.dot_general(a, b, (((1,), (1,)), ((), ())),
                           preferred_element_type=jnp.float32)


def _matmul_t(x, w):
    def body(x_ref, w_ref, o_ref):
        o_ref[...] = _dot_t(x_ref[...], w_ref[...])

    return pl.pallas_call(
        body,
        grid=(N_PAD // _BM,),
        in_specs=[
            pl.BlockSpec((_BM, D), lambda i: (i, 0)),
            pl.BlockSpec((D, D), lambda i: (0, 0)),
        ],
        out_specs=pl.BlockSpec((_BM, D), lambda i: (i, 0)),
        out_shape=jax.ShapeDtypeStruct((N_PAD, D), jnp.float32),
    )(x, w)


def _combine_mid(p0, p1, x, w_root, b, w2_rel):
    def body(p0_ref, p1_ref, x_ref, wr_ref, b_ref, w2_ref, h_ref, m2_ref):
        h = p0_ref[...] + p1_ref[...] + b_ref[...] + _dot_t(
            x_ref[...], wr_ref[...])
        h = jnp.maximum(h, 0.0)
        h_ref[...] = h
        m2_ref[...] = _dot_t(h, w2_ref[...])

    return pl.pallas_call(
        body,
        grid=(N_PAD // _BM,),
        in_specs=[
            pl.BlockSpec((_BM, D), lambda i: (i, 0)),
            pl.BlockSpec((_BM, D), lambda i: (i, 0)),
            pl.BlockSpec((_BM, D), lambda i: (i, 0)),
            pl.BlockSpec((D, D), lambda i: (0, 0)),
            pl.BlockSpec((1, D), lambda i: (0, 0)),
            pl.BlockSpec((D, D), lambda i: (0, 0)),
        ],
        out_specs=[
            pl.BlockSpec((_BM, D), lambda i: (i, 0)),
            pl.BlockSpec((_BM, D), lambda i: (i, 0)),
        ],
        out_shape=[
            jax.ShapeDtypeStruct((N_PAD, D), jnp.float32),
            jax.ShapeDtypeStruct((N_PAD, D), jnp.float32),
        ],
    )(p0, p1, x, w_root, b, w2_rel)


def _combine_last(p0, p1, h, w_root, b):
    def body(p0_ref, p1_ref, h_ref, wr_ref, b_ref, o_ref):
        o_ref[...] = p0_ref[...] + p1_ref[...] + b_ref[...] + _dot_t(
            h_ref[...], wr_ref[...])

    return pl.pallas_call(
        body,
        grid=(N_PAD // _BM,),
        in_specs=[
            pl.BlockSpec((_BM, D), lambda i: (i, 0)),
            pl.BlockSpec((_BM, D), lambda i: (i, 0)),
            pl.BlockSpec((_BM, D), lambda i: (i, 0)),
            pl.BlockSpec((D, D), lambda i: (0, 0)),
            pl.BlockSpec((1, D), lambda i: (0, 0)),
        ],
        out_specs=pl.BlockSpec((_BM, D), lambda i: (i, 0)),
        out_shape=jax.ShapeDtypeStruct((N_PAD, D), jnp.float32),
    )(p0, p1, h, w_root, b)


def kernel(x, edge_index, W1_rel, b1_rel, W1_root, W2_rel, b2_rel, W2_root):
    x_pad = jnp.pad(x, ((0, N_PAD - N), (0, 0)))
    pad = jnp.arange(E_PAD - E, dtype=jnp.int32)
    src = jnp.concatenate([edge_index[0], pad % N]).reshape(NW * K, G)
    dst = jnp.concatenate([edge_index[1],
                           N + pad % (N_PAD - N)]).reshape(NW * K, G)

    m1 = _matmul_t(x_pad, W1_rel)
    p1 = _segment_sum_sc(m1, src, dst)
    h1, m2 = _combine_mid(p1[0], p1[1], x_pad, W1_root,
                          b1_rel.reshape(1, D), W2_rel)
    p2 = _segment_sum_sc(m2, src, dst)
    out = _combine_last(p2[0], p2[1], h1, W2_root, b2_rel.reshape(1, D))
    return out[:N]


# speedup vs baseline: 1.0931x; 1.0931x over previous
import functools

import jax
import jax.numpy as jnp
from jax import lax
from jax.experimental import pallas as pl
from jax.experimental.pallas import tpu as pltpu
from jax.experimental.pallas import tpu_sc as plsc

N = 10000
E = 320000
D = 128

NC = 2
NS = 16
NW = NC * NS

G = 64
K = 160
CK = 16
E_PAD = NW * K * G
N_PAD = 10240
ROWS_PER_TILE = N_PAD // NS


NB = 4
NCHUNK = K // CK


def _segment_sum_sc(m_pad, src_r, dst_r):
    mesh = plsc.VectorSubcoreMesh(core_axis_name="c", subcore_axis_name="s")

    @functools.partial(
        pl.kernel,
        out_type=jax.ShapeDtypeStruct((NC, N_PAD, D), jnp.float32),
        mesh=mesh,
        scratch_types=[
            pltpu.VMEM((CK, G), jnp.int32),
            pltpu.VMEM((CK, G), jnp.int32),
            pltpu.VMEM((CK, G), jnp.int32),
            pltpu.VMEM((CK, G), jnp.int32),
            pltpu.VMEM((G, D), jnp.float32),
            pltpu.VMEM((G, D), jnp.float32),
            pltpu.VMEM((G, D), jnp.float32),
            pltpu.VMEM((G, D), jnp.float32),
            pltpu.VMEM_SHARED((N_PAD, D), jnp.float32),
            pltpu.SemaphoreType.DMA,
            pltpu.SemaphoreType.DMA,
            pltpu.SemaphoreType.DMA,
            pltpu.SemaphoreType.DMA,
            pltpu.SemaphoreType.DMA,
            pltpu.SemaphoreType.DMA,
            pltpu.SemaphoreType.DMA,
            pltpu.SemaphoreType.DMA,
            pltpu.SemaphoreType.DMA,
            pltpu.SemaphoreType.DMA,
        ],
    )
    def k(m_hbm, src_hbm, dst_hbm, out_hbm,
          src0, dst0, src1, dst1, b0, b1, b2, b3, acc,
          sg0, sg1, sg2, sg3, ss0, ss1, ss2, ss3, si, sz):
        bufs = [b0, b1, b2, b3]
        sg = [sg0, sg1, sg2, sg3]
        ss = [ss0, ss1, ss2, ss3]
        idx = [(src0, dst0), (src1, dst1)]

        cid = lax.axis_index("c")
        sid = lax.axis_index("s")
        wid = cid * NS + sid
        row0 = sid * ROWS_PER_TILE

        def gather(p, row, b):
            pltpu.async_copy(m_hbm.at[idx[p][0].at[row]], bufs[b], sg[b])

        def wait_gather(p, row, b):
            pltpu.make_async_copy(m_hbm.at[idx[p][0].at[row]], bufs[b],
                                  sg[b]).wait()

        def scatter(p, row, b):
            pltpu.async_copy(bufs[b], acc.at[idx[p][1].at[row]], ss[b],
                             add=True)

        def wait_scatter(p, row, b):
            pltpu.make_async_copy(bufs[b], acc.at[idx[p][1].at[row]],
                                  ss[b]).wait()

        def stage(c, p, sem):
            pltpu.async_copy(src_hbm.at[pl.ds(wid * K + c * CK, CK)],
                             idx[p][0], sem)
            pltpu.async_copy(dst_hbm.at[pl.ds(wid * K + c * CK, CK)],
                             idx[p][1], sem)

        def wait_stage(c, p, sem):
            pltpu.make_async_copy(src_hbm.at[pl.ds(wid * K + c * CK, CK)],
                                  idx[p][0], sem).wait()
            pltpu.make_async_copy(dst_hbm.at[pl.ds(wid * K + c * CK, CK)],
                                  idx[p][1], sem).wait()

        @pl.loop(0, G)
        def _(i):
            @pl.loop(0, D, step=16)
            def _(j):
                b0.at[pl.ds(i, 1), pl.ds(j, 16)][...] = jnp.zeros(
                    (1, 16), jnp.float32)

        for r in range(0, ROWS_PER_TILE, G):
            pltpu.async_copy(b0, acc.at[pl.ds(row0 + r, G)], sz)
        for r in range(0, ROWS_PER_TILE, G):
            pltpu.make_async_copy(b0, acc.at[pl.ds(row0 + r, G)], sz).wait()

        plsc.subcore_barrier()

        stage(0, 0, si)
        wait_stage(0, 0, si)

        for c in range(NCHUNK):
            p = c % 2
            q = 1 - p

            for i in range(4):
                if c > 0:
                    wait_scatter(q, CK - 4 + i, i)
                gather(p, i, i)
                if c > 0 or i >= 2:
                    if i < 2:
                        wait_gather(q, CK - 2 + i, (i + 2) % 4)
                        scatter(q, CK - 2 + i, (i + 2) % 4)
                    else:
                        wait_gather(p, i - 2, i - 2)
                        scatter(p, i - 2, i - 2)

            if c + 1 < NCHUNK:
                stage(c + 1, q, si)

            @pl.loop(4, CK, step=4)
            def _(j):
                for i in range(4):
                    wait_scatter(p, j + i - 4, i)
                    gather(p, j + i, i)
                    wait_gather(p, j + i - 2, (i + 2) % 4)
                    scatter(p, j + i - 2, (i + 2) % 4)

            if c + 1 < NCHUNK:
                wait_stage(c + 1, q, si)

        pl_last = (NCHUNK - 1) % 2
        wait_gather(pl_last, CK - 2, 2)
        scatter(pl_last, CK - 2, 2)
        wait_gather(pl_last, CK - 1, 3)
        scatter(pl_last, CK - 1, 3)
        for b in range(4):
            wait_scatter(pl_last, CK - 4 + b, b)

        plsc.subcore_barrier()

        for r in range(0, ROWS_PER_TILE, G):
            pltpu.async_copy(acc.at[pl.ds(row0 + r, G)],
                             out_hbm.at[cid, pl.ds(row0 + r, G)], sz)
        for r in range(0, ROWS_PER_TILE, G):
            pltpu.make_async_copy(acc.at[pl.ds(row0 + r, G)],
                                  out_hbm.at[cid, pl.ds(row0 + r, G)],
                                  sz).wait()

    return k(m_pad, src_r, dst_r)


_BM = 256


def _dot_t(a, b):
    return lax.dot_general(a, b, (((1,), (1,)), ((), ())),
                           preferred_element_type=jnp.float32)


def _matmul_t(x, w):
    def body(x_ref, w_ref, o_ref):
        o_ref[...] = _dot_t(x_ref[...], w_ref[...])

    return pl.pallas_call(
        body,
        grid=(N_PAD // _BM,),
        in_specs=[
            pl.BlockSpec((_BM, D), lambda i: (i, 0)),
            pl.BlockSpec((D, D), lambda i: (0, 0)),
        ],
        out_specs=pl.BlockSpec((_BM, D), lambda i: (i, 0)),
        out_shape=jax.ShapeDtypeStruct((N_PAD, D), jnp.float32),
    )(x, w)


def _combine_mid(p0, p1, x, w_root, b, w2_rel):
    def body(p0_ref, p1_ref, x_ref, wr_ref, b_ref, w2_ref, h_ref, m2_ref):
        h = p0_ref[...] + p1_ref[...] + b_ref[...] + _dot_t(
            x_ref[...], wr_ref[...])
        h = jnp.maximum(h, 0.0)
        h_ref[...] = h
        m2_ref[...] = _dot_t(h, w2_ref[...])

    return pl.pallas_call(
        body,
        grid=(N_PAD // _BM,),
        in_specs=[
            pl.BlockSpec((_BM, D), lambda i: (i, 0)),
            pl.BlockSpec((_BM, D), lambda i: (i, 0)),
            pl.BlockSpec((_BM, D), lambda i: (i, 0)),
            pl.BlockSpec((D, D), lambda i: (0, 0)),
            pl.BlockSpec((1, D), lambda i: (0, 0)),
            pl.BlockSpec((D, D), lambda i: (0, 0)),
        ],
        out_specs=[
            pl.BlockSpec((_BM, D), lambda i: (i, 0)),
            pl.BlockSpec((_BM, D), lambda i: (i, 0)),
        ],
        out_shape=[
            jax.ShapeDtypeStruct((N_PAD, D), jnp.float32),
            jax.ShapeDtypeStruct((N_PAD, D), jnp.float32),
        ],
    )(p0, p1, x, w_root, b, w2_rel)


def _combine_last(p0, p1, h, w_root, b):
    def body(p0_ref, p1_ref, h_ref, wr_ref, b_ref, o_ref):
        o_ref[...] = p0_ref[...] + p1_ref[...] + b_ref[...] + _dot_t(
            h_ref[...], wr_ref[...])

    return pl.pallas_call(
        body,
        grid=(N_PAD // _BM,),
        in_specs=[
            pl.BlockSpec((_BM, D), lambda i: (i, 0)),
            pl.BlockSpec((_BM, D), lambda i: (i, 0)),
            pl.BlockSpec((_BM, D), lambda i: (i, 0)),
            pl.BlockSpec((D, D), lambda i: (0, 0)),
            pl.BlockSpec((1, D), lambda i: (0, 0)),
        ],
        out_specs=pl.BlockSpec((_BM, D), lambda i: (i, 0)),
        out_shape=jax.ShapeDtypeStruct((N_PAD, D), jnp.float32),
    )(p0, p1, h, w_root, b)


def kernel(x, edge_index, W1_rel, b1_rel, W1_root, W2_rel, b2_rel, W2_root):
    x_pad = jnp.pad(x, ((0, N_PAD - N), (0, 0)))
    pad = jnp.arange(E_PAD - E, dtype=jnp.int32)
    src = jnp.concatenate([edge_index[0], pad % N]).reshape(NW * K, G)
    dst = jnp.concatenate([edge_index[1],
                           N + pad % (N_PAD - N)]).reshape(NW * K, G)

    m1 = _matmul_t(x_pad, W1_rel)
    p1 = _segment_sum_sc(m1, src, dst)
    h1, m2 = _combine_mid(p1[0], p1[1], x_pad, W1_root,
                          b1_rel.reshape(1, D), W2_rel)
    p2 = _segment_sum_sc(m2, src, dst)
    out = _combine_last(p2[0], p2[1], h1, W2_root, b2_rel.reshape(1, D))
    return out[:N]


# speedup vs baseline: 1.1059x; 1.0118x over previous
import functools

import jax
import jax.numpy as jnp
from jax import lax
from jax.experimental import pallas as pl
from jax.experimental.pallas import tpu as pltpu
from jax.experimental.pallas import tpu_sc as plsc

N = 10000
E = 320000
D = 128

NC = 2
NS = 16
NW = NC * NS

G = 64
K = 160
CK = 16
E_PAD = NW * K * G
N_PAD = 10240
ROWS_PER_TILE = N_PAD // NS


NB = 4
NCHUNK = K // CK


def _segment_sum_sc(m_pad, src_r, dst_r):
    mesh = plsc.VectorSubcoreMesh(core_axis_name="c", subcore_axis_name="s")

    @functools.partial(
        pl.kernel,
        out_type=jax.ShapeDtypeStruct((NC, N_PAD, D), jnp.float32),
        mesh=mesh,
        scratch_types=[
            pltpu.VMEM((CK, G), jnp.int32),
            pltpu.VMEM((CK, G), jnp.int32),
            pltpu.VMEM((CK, G), jnp.int32),
            pltpu.VMEM((CK, G), jnp.int32),
            pltpu.VMEM((G, D), jnp.float32),
            pltpu.VMEM((G, D), jnp.float32),
            pltpu.VMEM((G, D), jnp.float32),
            pltpu.VMEM((G, D), jnp.float32),
            pltpu.VMEM_SHARED((N_PAD, D), jnp.float32),
            pltpu.SemaphoreType.DMA,
            pltpu.SemaphoreType.DMA,
            pltpu.SemaphoreType.DMA,
            pltpu.SemaphoreType.DMA,
            pltpu.SemaphoreType.DMA,
            pltpu.SemaphoreType.DMA,
            pltpu.SemaphoreType.DMA,
            pltpu.SemaphoreType.DMA,
            pltpu.SemaphoreType.DMA,
            pltpu.SemaphoreType.DMA,
        ],
    )
    def k(m_hbm, src_hbm, dst_hbm, out_hbm,
          src0, dst0, src1, dst1, b0, b1, b2, b3, acc,
          sg0, sg1, sg2, sg3, ss0, ss1, ss2, ss3, si, sz):
        bufs = [b0, b1, b2, b3]
        sg = [sg0, sg1, sg2, sg3]
        ss = [ss0, ss1, ss2, ss3]
        idx = [(src0, dst0), (src1, dst1)]

        cid = lax.axis_index("c")
        sid = lax.axis_index("s")
        wid = cid * NS + sid
        row0 = sid * ROWS_PER_TILE

        def gather(p, row, b):
            pltpu.async_copy(m_hbm.at[idx[p][0].at[row]], bufs[b], sg[b])

        def wait_gather(p, row, b):
            pltpu.make_async_copy(m_hbm.at[idx[p][0].at[row]], bufs[b],
                                  sg[b]).wait()

        def scatter(p, row, b):
            pltpu.async_copy(bufs[b], acc.at[idx[p][1].at[row]], ss[b],
                             add=True)

        def wait_scatter(p, row, b):
            pltpu.make_async_copy(bufs[b], acc.at[idx[p][1].at[row]],
                                  ss[b]).wait()

        def stage(c, p, sem):
            pltpu.async_copy(src_hbm.at[pl.ds(wid * K + c * CK, CK)],
                             idx[p][0], sem)
            pltpu.async_copy(dst_hbm.at[pl.ds(wid * K + c * CK, CK)],
                             idx[p][1], sem)

        def wait_stage(c, p, sem):
            pltpu.make_async_copy(src_hbm.at[pl.ds(wid * K + c * CK, CK)],
                                  idx[p][0], sem).wait()
            pltpu.make_async_copy(dst_hbm.at[pl.ds(wid * K + c * CK, CK)],
                                  idx[p][1], sem).wait()

        @pl.loop(0, G)
        def _(i):
            @pl.loop(0, D, step=16)
            def _(j):
                b0.at[pl.ds(i, 1), pl.ds(j, 16)][...] = jnp.zeros(
                    (1, 16), jnp.float32)

        for r in range(0, ROWS_PER_TILE, G):
            pltpu.async_copy(b0, acc.at[pl.ds(row0 + r, G)], sz)
        for r in range(0, ROWS_PER_TILE, G):
            pltpu.make_async_copy(b0, acc.at[pl.ds(row0 + r, G)], sz).wait()

        plsc.subcore_barrier()

        stage(0, 0, si)
        wait_stage(0, 0, si)

        for c in range(NCHUNK):
            p = c % 2
            q = 1 - p

            for i in range(4):
                if c > 0:
                    wait_scatter(q, CK - 4 + i, i)
                gather(p, i, i)
                if c > 0 or i >= 2:
                    if i < 2:
                        wait_gather(q, CK - 2 + i, (i + 2) % 4)
                        scatter(q, CK - 2 + i, (i + 2) % 4)
                    else:
                        wait_gather(p, i - 2, i - 2)
                        scatter(p, i - 2, i - 2)

            if c + 1 < NCHUNK:
                stage(c + 1, q, si)

            @pl.loop(4, CK, step=4)
            def _(j):
                for i in range(4):
                    wait_scatter(p, j + i - 4, i)
                    gather(p, j + i, i)
                    wait_gather(p, j + i - 2, (i + 2) % 4)
                    scatter(p, j + i - 2, (i + 2) % 4)

            if c + 1 < NCHUNK:
                wait_stage(c + 1, q, si)

        pl_last = (NCHUNK - 1) % 2
        wait_gather(pl_last, CK - 2, 2)
        scatter(pl_last, CK - 2, 2)
        wait_gather(pl_last, CK - 1, 3)
        scatter(pl_last, CK - 1, 3)
        for b in range(4):
            wait_scatter(pl_last, CK - 4 + b, b)

        plsc.subcore_barrier()

        for r in range(0, ROWS_PER_TILE, G):
            pltpu.async_copy(acc.at[pl.ds(row0 + r, G)],
                             out_hbm.at[cid, pl.ds(row0 + r, G)], sz)
        for r in range(0, ROWS_PER_TILE, G):
            pltpu.make_async_copy(acc.at[pl.ds(row0 + r, G)],
                                  out_hbm.at[cid, pl.ds(row0 + r, G)],
                                  sz).wait()

    return k(m_pad, src_r, dst_r)


_BM = 256


def _dot_t(a, b):
    return lax.dot_general(a, b, (((1,), (1,)), ((), ())),
                           preferred_element_type=jnp.float32)


def _matmul_t(x, w):
    def body(x_ref, w_ref, o_ref):
        o_ref[...] = _dot_t(x_ref[...], w_ref[...])

    return pl.pallas_call(
        body,
        grid=(N_PAD // _BM,),
        in_specs=[
            pl.BlockSpec((_BM, D), lambda i: (i, 0)),
            pl.BlockSpec((D, D), lambda i: (0, 0)),
        ],
        out_specs=pl.BlockSpec((_BM, D), lambda i: (i, 0)),
        out_shape=jax.ShapeDtypeStruct((N_PAD, D), jnp.float32),
    )(x, w)


def _matmul_bias_t(x, w, b):
    def body(x_ref, w_ref, b_ref, o_ref):
        o_ref[...] = _dot_t(x_ref[...], w_ref[...]) + b_ref[...]

    return pl.pallas_call(
        body,
        grid=(N_PAD // _BM,),
        in_specs=[
            pl.BlockSpec((_BM, D), lambda i: (i, 0)),
            pl.BlockSpec((D, D), lambda i: (0, 0)),
            pl.BlockSpec((1, D), lambda i: (0, 0)),
        ],
        out_specs=pl.BlockSpec((_BM, D), lambda i: (i, 0)),
        out_shape=jax.ShapeDtypeStruct((N_PAD, D), jnp.float32),
    )(x, w, b)


def _combine_mid(p0, p1, root1, w2_rel):
    def body(p0_ref, p1_ref, r_ref, w2_ref, h_ref, m2_ref):
        h = jnp.maximum(p0_ref[...] + p1_ref[...] + r_ref[...], 0.0)
        h_ref[...] = h
        m2_ref[...] = _dot_t(h, w2_ref[...])

    return pl.pallas_call(
        body,
        grid=(N_PAD // _BM,),
        in_specs=[
            pl.BlockSpec((_BM, D), lambda i: (i, 0)),
            pl.BlockSpec((_BM, D), lambda i: (i, 0)),
            pl.BlockSpec((_BM, D), lambda i: (i, 0)),
            pl.BlockSpec((D, D), lambda i: (0, 0)),
        ],
        out_specs=[
            pl.BlockSpec((_BM, D), lambda i: (i, 0)),
            pl.BlockSpec((_BM, D), lambda i: (i, 0)),
        ],
        out_shape=[
            jax.ShapeDtypeStruct((N_PAD, D), jnp.float32),
            jax.ShapeDtypeStruct((N_PAD, D), jnp.float32),
        ],
    )(p0, p1, root1, w2_rel)


def _combine_last(p0, p1, root2):
    def body(p0_ref, p1_ref, r_ref, o_ref):
        o_ref[...] = p0_ref[...] + p1_ref[...] + r_ref[...]

    return pl.pallas_call(
        body,
        grid=(N_PAD // _BM,),
        in_specs=[
            pl.BlockSpec((_BM, D), lambda i: (i, 0)),
            pl.BlockSpec((_BM, D), lambda i: (i, 0)),
            pl.BlockSpec((_BM, D), lambda i: (i, 0)),
        ],
        out_specs=pl.BlockSpec((_BM, D), lambda i: (i, 0)),
        out_shape=jax.ShapeDtypeStruct((N_PAD, D), jnp.float32),
    )(p0, p1, root2)


def kernel(x, edge_index, W1_rel, b1_rel, W1_root, W2_rel, b2_rel, W2_root):
    x_pad = jnp.pad(x, ((0, N_PAD - N), (0, 0)))
    pad = jnp.arange(E_PAD - E, dtype=jnp.int32)
    src = jnp.concatenate([edge_index[0], pad % N]).reshape(NW * K, G)
    dst = jnp.concatenate([edge_index[1],
                           N + pad % (N_PAD - N)]).reshape(NW * K, G)

    m1 = _matmul_t(x_pad, W1_rel)
    p1 = _segment_sum_sc(m1, src, dst)
    root1 = _matmul_bias_t(x_pad, W1_root, b1_rel.reshape(1, D))
    h1, m2 = _combine_mid(p1[0], p1[1], root1, W2_rel)
    p2 = _segment_sum_sc(m2, src, dst)
    root2 = _matmul_bias_t(h1, W2_root, b2_rel.reshape(1, D))
    out = _combine_last(p2[0], p2[1], root2)
    return out[:N]


# speedup vs baseline: 1.1734x; 1.0610x over previous
import functools

import jax
import jax.numpy as jnp
from jax import lax
from jax.experimental import pallas as pl
from jax.experimental.pallas import tpu as pltpu
from jax.experimental.pallas import tpu_sc as plsc

N = 10000
E = 320000
D = 128

NC = 2
NS = 16
NW = NC * NS

G = 64
K = 160
CK = 16
E_PAD = NW * K * G
N_PAD = 10240
ROWS_PER_TILE = N_PAD // NS


NB = 4
NCHUNK = K // CK


def _segment_sum_sc(m_pad, src_r, dst_r):
    mesh = plsc.VectorSubcoreMesh(core_axis_name="c", subcore_axis_name="s")

    @functools.partial(
        pl.kernel,
        out_type=jax.ShapeDtypeStruct((NC, N_PAD, D), jnp.float32),
        mesh=mesh,
        scratch_types=[
            pltpu.VMEM((CK, G), jnp.int32),
            pltpu.VMEM((CK, G), jnp.int32),
            pltpu.VMEM((CK, G), jnp.int32),
            pltpu.VMEM((CK, G), jnp.int32),
            pltpu.VMEM((G, D), jnp.float32),
            pltpu.VMEM((G, D), jnp.float32),
            pltpu.VMEM((G, D), jnp.float32),
            pltpu.VMEM((G, D), jnp.float32),
            pltpu.VMEM_SHARED((N_PAD, D), jnp.float32),
            pltpu.SemaphoreType.DMA,
            pltpu.SemaphoreType.DMA,
            pltpu.SemaphoreType.DMA,
            pltpu.SemaphoreType.DMA,
            pltpu.SemaphoreType.DMA,
            pltpu.SemaphoreType.DMA,
            pltpu.SemaphoreType.DMA,
            pltpu.SemaphoreType.DMA,
            pltpu.SemaphoreType.DMA,
            pltpu.SemaphoreType.DMA,
        ],
    )
    def k(m_hbm, src_hbm, dst_hbm, out_hbm,
          src0, dst0, src1, dst1, b0, b1, b2, b3, acc,
          sg0, sg1, sg2, sg3, ss0, ss1, ss2, ss3, si, sz):
        bufs = [b0, b1, b2, b3]
        sg = [sg0, sg1, sg2, sg3]
        ss = [ss0, ss1, ss2, ss3]
        idx = [(src0, dst0), (src1, dst1)]

        cid = lax.axis_index("c")
        sid = lax.axis_index("s")
        wid = cid * NS + sid
        row0 = sid * ROWS_PER_TILE

        def gather(p, row, b):
            pltpu.async_copy(m_hbm.at[idx[p][0].at[row]], bufs[b], sg[b])

        def wait_gather(p, row, b):
            pltpu.make_async_copy(m_hbm.at[idx[p][0].at[row]], bufs[b],
                                  sg[b]).wait()

        def scatter(p, row, b):
            pltpu.async_copy(bufs[b], acc.at[idx[p][1].at[row]], ss[b],
                             add=True)

        def wait_scatter(p, row, b):
            pltpu.make_async_copy(bufs[b], acc.at[idx[p][1].at[row]],
                                  ss[b]).wait()

        def stage(c, p, sem):
            pltpu.async_copy(src_hbm.at[pl.ds(wid * K + c * CK, CK)],
                             idx[p][0], sem)
            pltpu.async_copy(dst_hbm.at[pl.ds(wid * K + c * CK, CK)],
                             idx[p][1], sem)

        def wait_stage(c, p, sem):
            pltpu.make_async_copy(src_hbm.at[pl.ds(wid * K + c * CK, CK)],
                                  idx[p][0], sem).wait()
            pltpu.make_async_copy(dst_hbm.at[pl.ds(wid * K + c * CK, CK)],
                                  idx[p][1], sem).wait()

        @pl.loop(0, G)
        def _(i):
            @pl.loop(0, D, step=16)
            def _(j):
                b0.at[pl.ds(i, 1), pl.ds(j, 16)][...] = jnp.zeros(
                    (1, 16), jnp.float32)

        for r in range(0, ROWS_PER_TILE, G):
            pltpu.async_copy(b0, acc.at[pl.ds(row0 + r, G)], sz)
        for r in range(0, ROWS_PER_TILE, G):
            pltpu.make_async_copy(b0, acc.at[pl.ds(row0 + r, G)], sz).wait()

        plsc.subcore_barrier()

        stage(0, 0, si)
        wait_stage(0, 0, si)

        for c in range(NCHUNK):
            p = c % 2
            q = 1 - p

            for i in range(4):
                if c > 0:
                    wait_scatter(q, CK - 4 + i, i)
                gather(p, i, i)
                if c > 0 or i >= 3:
                    if i < 3:
                        wait_gather(q, CK - 3 + i, (i + 1) % 4)
                        scatter(q, CK - 3 + i, (i + 1) % 4)
                    else:
                        wait_gather(p, i - 3, i - 3)
                        scatter(p, i - 3, i - 3)

            if c + 1 < NCHUNK:
                stage(c + 1, q, si)

            @pl.loop(4, CK, step=4)
            def _(j):
                for i in range(4):
                    wait_scatter(p, j + i - 4, i)
                    gather(p, j + i, i)
                    wait_gather(p, j + i - 3, (i + 1) % 4)
                    scatter(p, j + i - 3, (i + 1) % 4)

            if c + 1 < NCHUNK:
                wait_stage(c + 1, q, si)

        pl_last = (NCHUNK - 1) % 2
        for r in range(CK - 3, CK):
            b = (r % 4)
            wait_gather(pl_last, r, b)
            scatter(pl_last, r, b)
        for b in range(4):
            wait_scatter(pl_last, CK - 4 + b, b)

        plsc.subcore_barrier()

        for r in range(0, ROWS_PER_TILE, G):
            pltpu.async_copy(acc.at[pl.ds(row0 + r, G)],
                             out_hbm.at[cid, pl.ds(row0 + r, G)], sz)
        for r in range(0, ROWS_PER_TILE, G):
            pltpu.make_async_copy(acc.at[pl.ds(row0 + r, G)],
                                  out_hbm.at[cid, pl.ds(row0 + r, G)],
                                  sz).wait()

    return k(m_pad, src_r, dst_r)


_BM = 256


def _dot_t(a, b):
    return lax.dot_general(a, b, (((1,), (1,)), ((), ())),
                           preferred_element_type=jnp.float32)


def _matmul_t(x, w):
    def body(x_ref, w_ref, o_ref):
        o_ref[...] = _dot_t(x_ref[...], w_ref[...])

    return pl.pallas_call(
        body,
        grid=(N_PAD // _BM,),
        in_specs=[
            pl.BlockSpec((_BM, D), lambda i: (i, 0)),
            pl.BlockSpec((D, D), lambda i: (0, 0)),
        ],
        out_specs=pl.BlockSpec((_BM, D), lambda i: (i, 0)),
        out_shape=jax.ShapeDtypeStruct((N_PAD, D), jnp.float32),
    )(x, w)


def _matmul_bias_t(x, w, b):
    def body(x_ref, w_ref, b_ref, o_ref):
        o_ref[...] = _dot_t(x_ref[...], w_ref[...]) + b_ref[...]

    return pl.pallas_call(
        body,
        grid=(N_PAD // _BM,),
        in_specs=[
            pl.BlockSpec((_BM, D), lambda i: (i, 0)),
            pl.BlockSpec((D, D), lambda i: (0, 0)),
            pl.BlockSpec((1, D), lambda i: (0, 0)),
        ],
        out_specs=pl.BlockSpec((_BM, D), lambda i: (i, 0)),
        out_shape=jax.ShapeDtypeStruct((N_PAD, D), jnp.float32),
    )(x, w, b)


def _combine_mid(p0, p1, root1, w2_rel):
    def body(p0_ref, p1_ref, r_ref, w2_ref, h_ref, m2_ref):
        h = jnp.maximum(p0_ref[...] + p1_ref[...] + r_ref[...], 0.0)
        h_ref[...] = h
        m2_ref[...] = _dot_t(h, w2_ref[...])

    return pl.pallas_call(
        body,
        grid=(N_PAD // _BM,),
        in_specs=[
            pl.BlockSpec((_BM, D), lambda i: (i, 0)),
            pl.BlockSpec((_BM, D), lambda i: (i, 0)),
            pl.BlockSpec((_BM, D), lambda i: (i, 0)),
            pl.BlockSpec((D, D), lambda i: (0, 0)),
        ],
        out_specs=[
            pl.BlockSpec((_BM, D), lambda i: (i, 0)),
            pl.BlockSpec((_BM, D), lambda i: (i, 0)),
        ],
        out_shape=[
            jax.ShapeDtypeStruct((N_PAD, D), jnp.float32),
            jax.ShapeDtypeStruct((N_PAD, D), jnp.float32),
        ],
    )(p0, p1, root1, w2_rel)


def _combine_last(p0, p1, root2):
    def body(p0_ref, p1_ref, r_ref, o_ref):
        o_ref[...] = p0_ref[...] + p1_ref[...] + r_ref[...]

    return pl.pallas_call(
        body,
        grid=(N_PAD // _BM,),
        in_specs=[
            pl.BlockSpec((_BM, D), lambda i: (i, 0)),
            pl.BlockSpec((_BM, D), lambda i: (i, 0)),
            pl.BlockSpec((_BM, D), lambda i: (i, 0)),
        ],
        out_specs=pl.BlockSpec((_BM, D), lambda i: (i, 0)),
        out_shape=jax.ShapeDtypeStruct((N_PAD, D), jnp.float32),
    )(p0, p1, root2)


def kernel(x, edge_index, W1_rel, b1_rel, W1_root, W2_rel, b2_rel, W2_root):
    x_pad = jnp.pad(x, ((0, N_PAD - N), (0, 0)))
    pad = jnp.arange(E_PAD - E, dtype=jnp.int32)
    src = jnp.concatenate([edge_index[0], pad % N]).reshape(NW * K, G)
    dst = jnp.concatenate([edge_index[1],
                           N + pad % (N_PAD - N)]).reshape(NW * K, G)

    m1 = _matmul_t(x_pad, W1_rel)
    p1 = _segment_sum_sc(m1, src, dst)
    root1 = _matmul_bias_t(x_pad, W1_root, b1_rel.reshape(1, D))
    h1, m2 = _combine_mid(p1[0], p1[1], root1, W2_rel)
    p2 = _segment_sum_sc(m2, src, dst)
    root2 = _matmul_bias_t(h1, W2_root, b2_rel.reshape(1, D))
    out = _combine_last(p2[0], p2[1], root2)
    return out[:N]
